# Initial kernel scaffold; baseline (speedup 1.0000x reference)
import functools

import jax
import jax.numpy as jnp
from jax import lax
from jax.experimental import pallas as pl
from jax.experimental.pallas import tpu as pltpu
from jax.experimental.pallas import tpu_sc as plsc

NC = 2
NS = 16
BLK = 125


def _sc_aggregate(nbr_feat, idx2d, n_nodes, nblk_per_tile, deg_stripe):
    E, D = nbr_feat.shape
    NW = NC * NS
    e_per_tile = E // NW
    rows_per_tile = n_nodes // NS
    n_zero_blocks = rows_per_tile // BLK
    deg_pad = NS * deg_stripe

    mesh = plsc.VectorSubcoreMesh(core_axis_name="c", subcore_axis_name="s")

    @functools.partial(
        pl.kernel,
        mesh=mesh,
        out_type=[
            jax.ShapeDtypeStruct((NC, n_nodes, D), jnp.float32),
            jax.ShapeDtypeStruct((NC, deg_pad), jnp.float32),
        ],
        scratch_types=[
            pltpu.VMEM((nblk_per_tile, BLK), jnp.int32),
            pltpu.VMEM((BLK, D), jnp.float32),
            pltpu.VMEM((BLK, D), jnp.float32),
            pltpu.VMEM((128,), jnp.float32),
            pltpu.VMEM_SHARED((n_nodes, D), jnp.float32),
            pltpu.VMEM_SHARED((deg_pad,), jnp.float32),
        ],
    )
    def k(nbr_hbm, idx_hbm, out_sum, out_deg,
          idx_v, buf, zbuf, ones_v, acc_sh, deg_sh):
        c = lax.axis_index("c")
        s = lax.axis_index("s")
        wid = s * NC + c

        pltpu.sync_copy(idx_hbm.at[pl.ds(wid * nblk_per_tile, nblk_per_tile)],
                        idx_v)

        zero16 = jnp.zeros((16,), jnp.float32)
        one16 = jnp.ones((16,), jnp.float32)

        def fill_row(r, carry):
            for cc in range(D // 16):
                zbuf[r, pl.ds(cc * 16, 16)] = zero16
            return carry

        lax.fori_loop(0, BLK, fill_row, None)
        for i in range(128 // 16):
            ones_v[pl.ds(i * 16, 16)] = one16

        for j in range(n_zero_blocks):
            pltpu.sync_copy(
                zbuf, acc_sh.at[pl.ds(s * rows_per_tile + j * BLK, BLK)])
        for j in range(deg_stripe // 128):
            pltpu.sync_copy(
                zbuf.at[0], deg_sh.at[pl.ds(s * deg_stripe + j * 128, 128)])

        plsc.subcore_barrier()

        ebase = wid * e_per_tile

        def blk_body(b, carry):
            pltpu.sync_copy(nbr_hbm.at[pl.ds(ebase + b * BLK, BLK)], buf)
            pltpu.sync_copy(buf, acc_sh.at[idx_v.at[b]], add=True)
            pltpu.sync_copy(ones_v.at[pl.ds(0, BLK)],
                            deg_sh.at[idx_v.at[b]], add=True)
            return carry

        lax.fori_loop(0, nblk_per_tile, blk_body, None)

        plsc.subcore_barrier()

        pltpu.sync_copy(acc_sh.at[pl.ds(s * rows_per_tile, rows_per_tile)],
                        out_sum.at[c, pl.ds(s * rows_per_tile, rows_per_tile)])
        pltpu.sync_copy(deg_sh.at[pl.ds(s * deg_stripe, deg_stripe)],
                        out_deg.at[c, pl.ds(s * deg_stripe, deg_stripe)])

    return k(nbr_feat, idx2d)


def _tc_finish(self_feat, psum, pdeg, W):
    N, D = self_feat.shape

    def body(self_ref, p_ref, d_ref, w_ref, o_ref):
        x = self_ref[...] + p_ref[0] + p_ref[1]
        deg = d_ref[0] + d_ref[1] + 1.0
        y = x / deg
        o_ref[...] = lax.dot_general(
            y, w_ref[...],
            dimension_numbers=(((1,), (1,)), ((), ())),
            preferred_element_type=jnp.float32)

    return pl.pallas_call(
        body,
        out_shape=jax.ShapeDtypeStruct((N, D), jnp.float32),
    )(self_feat, psum, pdeg, W)


def kernel(self_feat, nbr_feat, relation_src_indices, W):
    N, D = self_feat.shape
    E = nbr_feat.shape[0]
    NW = NC * NS
    e_per_tile = E // NW
    nblk_per_tile = e_per_tile // BLK
    assert e_per_tile * NW == E and nblk_per_tile * BLK == e_per_tile
    assert N % (NS * BLK) == 0
    deg_stripe = -(-(N // NS) // 128) * 128

    idx2d = relation_src_indices.astype(jnp.int32).reshape(NW * nblk_per_tile,
                                                           BLK)
    psum, pdeg = _sc_aggregate(nbr_feat, idx2d, N, nblk_per_tile, deg_stripe)
    pdeg3 = pdeg[:, :N].reshape(NC, N, 1)
    return _tc_finish(self_feat, psum, pdeg3, W)


# speedup vs baseline: 8.4502x; 8.4502x over previous
import functools

import jax
import jax.numpy as jnp
from jax import lax
from jax.experimental import pallas as pl
from jax.experimental.pallas import tpu as pltpu
from jax.experimental.pallas import tpu_sc as plsc

NC = 2
NS = 16
BLK = 125


def _sc_aggregate(nbr_feat, idx2d, n_nodes, nblk_per_tile, deg_stripe):
    E, D = nbr_feat.shape
    NW = NC * NS
    e_per_tile = E // NW
    rows_per_tile = n_nodes // NS
    n_zero_blocks = rows_per_tile // BLK
    deg_pad = NS * deg_stripe

    mesh = plsc.VectorSubcoreMesh(core_axis_name="c", subcore_axis_name="s")

    @functools.partial(
        pl.kernel,
        mesh=mesh,
        compiler_params=pltpu.CompilerParams(use_tc_tiling_on_sc=False),
        out_type=[
            jax.ShapeDtypeStruct((NC, n_nodes, D), jnp.float32),
            jax.ShapeDtypeStruct((NC, deg_pad), jnp.float32),
        ],
        scratch_types=[
            pltpu.VMEM((nblk_per_tile, BLK), jnp.int32),
            pltpu.VMEM((BLK, D), jnp.float32),
            pltpu.VMEM((BLK, D), jnp.float32),
            pltpu.VMEM((128,), jnp.float32),
            pltpu.VMEM_SHARED((n_nodes, D), jnp.float32),
            pltpu.VMEM_SHARED((deg_pad,), jnp.float32),
        ],
    )
    def k(nbr_hbm, idx_hbm, out_sum, out_deg,
          idx_v, buf, zbuf, ones_v, acc_sh, deg_sh):
        c = lax.axis_index("c")
        s = lax.axis_index("s")
        wid = s * NC + c

        pltpu.sync_copy(idx_hbm.at[pl.ds(wid * nblk_per_tile, nblk_per_tile)],
                        idx_v)

        zero16 = jnp.zeros((16,), jnp.float32)
        one16 = jnp.ones((16,), jnp.float32)

        def fill_row(r, carry):
            for cc in range(D // 16):
                zbuf[r, pl.ds(cc * 16, 16)] = zero16
            return carry

        lax.fori_loop(0, BLK, fill_row, None)
        for i in range(128 // 16):
            ones_v[pl.ds(i * 16, 16)] = one16

        for j in range(n_zero_blocks):
            pltpu.sync_copy(
                zbuf, acc_sh.at[pl.ds(s * rows_per_tile + j * BLK, BLK)])
        for j in range(deg_stripe // 128):
            pltpu.sync_copy(
                zbuf.at[0], deg_sh.at[pl.ds(s * deg_stripe + j * 128, 128)])

        plsc.subcore_barrier()

        ebase = wid * e_per_tile

        def blk_body(b, carry):
            pltpu.sync_copy(nbr_hbm.at[pl.ds(ebase + b * BLK, BLK)], buf)
            pltpu.sync_copy(buf, acc_sh.at[idx_v.at[b]], add=True)
            pltpu.sync_copy(ones_v.at[pl.ds(0, BLK)],
                            deg_sh.at[idx_v.at[b]], add=True)
            return carry

        lax.fori_loop(0, nblk_per_tile, blk_body, None)

        plsc.subcore_barrier()

        pltpu.sync_copy(acc_sh.at[pl.ds(s * rows_per_tile, rows_per_tile)],
                        out_sum.at[c, pl.ds(s * rows_per_tile, rows_per_tile)])
        pltpu.sync_copy(deg_sh.at[pl.ds(s * deg_stripe, deg_stripe)],
                        out_deg.at[c, pl.ds(s * deg_stripe, deg_stripe)])

    return k(nbr_feat, idx2d)


def _tc_finish(self_feat, psum, pdeg, W):
    N, D = self_feat.shape

    def body(self_ref, p_ref, d_ref, w_ref, o_ref):
        x = self_ref[...] + p_ref[0] + p_ref[1]
        deg = d_ref[0] + d_ref[1] + 1.0
        y = x / deg
        o_ref[...] = lax.dot_general(
            y, w_ref[...],
            dimension_numbers=(((1,), (1,)), ((), ())),
            preferred_element_type=jnp.float32)

    return pl.pallas_call(
        body,
        out_shape=jax.ShapeDtypeStruct((N, D), jnp.float32),
    )(self_feat, psum, pdeg, W)


def kernel(self_feat, nbr_feat, relation_src_indices, W):
    N, D = self_feat.shape
    E = nbr_feat.shape[0]
    NW = NC * NS
    e_per_tile = E // NW
    nblk_per_tile = e_per_tile // BLK
    assert e_per_tile * NW == E and nblk_per_tile * BLK == e_per_tile
    assert N % (NS * BLK) == 0
    deg_stripe = -(-(N // NS) // 128) * 128

    idx2d = relation_src_indices.astype(jnp.int32).reshape(NW * nblk_per_tile,
                                                           BLK)
    psum, pdeg = _sc_aggregate(nbr_feat, idx2d, N, nblk_per_tile, deg_stripe)
    pdeg3 = pdeg[:, :N].reshape(NC, N, 1)
    return _tc_finish(self_feat, psum, pdeg3, W)


# speedup vs baseline: 11.2993x; 1.3372x over previous
import functools

import jax
import jax.numpy as jnp
from jax import lax
from jax.experimental import pallas as pl
from jax.experimental.pallas import tpu as pltpu
from jax.experimental.pallas import tpu_sc as plsc

NC = 2
NS = 16
BLK = 125


def _sc_aggregate(nbr_feat, idx2d, n_nodes, nblk_per_tile, deg_stripe):
    E, D = nbr_feat.shape
    NW = NC * NS
    e_per_tile = E // NW
    rows_per_tile = n_nodes // NS
    n_zero_blocks = rows_per_tile // BLK
    deg_pad = NS * deg_stripe

    mesh = plsc.VectorSubcoreMesh(core_axis_name="c", subcore_axis_name="s")

    nbuf = 2
    ngrp = nblk_per_tile // nbuf

    @functools.partial(
        pl.kernel,
        mesh=mesh,
        compiler_params=pltpu.CompilerParams(use_tc_tiling_on_sc=False),
        out_type=[
            jax.ShapeDtypeStruct((NC, n_nodes, D), jnp.float32),
            jax.ShapeDtypeStruct((NC, deg_pad), jnp.float32),
        ],
        scratch_types=[
            pltpu.VMEM((nblk_per_tile, BLK), jnp.int32),
            pltpu.VMEM((nbuf, BLK, D), jnp.float32),
            pltpu.VMEM((25, D), jnp.float32),
            pltpu.VMEM((128,), jnp.float32),
            pltpu.VMEM_SHARED((n_nodes, D), jnp.float32),
            pltpu.VMEM_SHARED((deg_pad,), jnp.float32),
            pltpu.SemaphoreType.DMA((nbuf,)),
            pltpu.SemaphoreType.DMA((nbuf,)),
            pltpu.SemaphoreType.DMA((nbuf,)),
        ],
    )
    def k(nbr_hbm, idx_hbm, out_sum, out_deg,
          idx_v, buf, zbuf, ones_v, acc_sh, deg_sh, sem_g, sem_s, sem_d):
        c = lax.axis_index("c")
        s = lax.axis_index("s")
        wid = s * NC + c

        pltpu.sync_copy(idx_hbm.at[pl.ds(wid * nblk_per_tile, nblk_per_tile)],
                        idx_v)

        zero16 = jnp.zeros((16,), jnp.float32)
        one16 = jnp.ones((16,), jnp.float32)

        def fill_row(r, carry):
            for cc in range(D // 16):
                zbuf[r, pl.ds(cc * 16, 16)] = zero16
            return carry

        lax.fori_loop(0, 25, fill_row, None)
        for i in range(128 // 16):
            ones_v[pl.ds(i * 16, 16)] = one16

        for j in range(rows_per_tile // 25):
            pltpu.sync_copy(
                zbuf, acc_sh.at[pl.ds(s * rows_per_tile + j * 25, 25)])
        for j in range(deg_stripe // 128):
            pltpu.sync_copy(
                zbuf.at[0], deg_sh.at[pl.ds(s * deg_stripe + j * 128, 128)])

        plsc.subcore_barrier()

        ebase = wid * e_per_tile

        for b in range(nbuf):
            pltpu.async_copy(nbr_hbm.at[pl.ds(ebase + b * BLK, BLK)],
                             buf.at[b], sem_g.at[b])

        def grp_body(g, carry):
            base_blk = g * nbuf
            feat_descs = []
            deg_descs = []
            for b in range(nbuf):
                blk = base_blk + b
                pltpu.make_async_copy(nbr_hbm.at[pl.ds(ebase, BLK)],
                                      buf.at[b], sem_g.at[b]).wait()
                feat_descs.append(pltpu.async_copy(
                    buf.at[b], acc_sh.at[idx_v.at[blk]], sem_s.at[b],
                    add=True))
                deg_descs.append(pltpu.async_copy(
                    ones_v.at[pl.ds(0, BLK)], deg_sh.at[idx_v.at[blk]],
                    sem_d.at[b], add=True))
            for b in range(nbuf):
                feat_descs[b].wait()
                deg_descs[b].wait()
                blk_next = base_blk + nbuf + b

                @pl.when(blk_next < nblk_per_tile)
                def _():
                    pltpu.async_copy(
                        nbr_hbm.at[pl.ds(ebase + blk_next * BLK, BLK)],
                        buf.at[b], sem_g.at[b])
            return carry

        lax.fori_loop(0, ngrp, grp_body, None)

        plsc.subcore_barrier()

        pltpu.sync_copy(acc_sh.at[pl.ds(s * rows_per_tile, rows_per_tile)],
                        out_sum.at[c, pl.ds(s * rows_per_tile, rows_per_tile)])
        pltpu.sync_copy(deg_sh.at[pl.ds(s * deg_stripe, deg_stripe)],
                        out_deg.at[c, pl.ds(s * deg_stripe, deg_stripe)])

    return k(nbr_feat, idx2d)


def _tc_finish(self_feat, psum, pdeg, W):
    N, D = self_feat.shape

    def body(self_ref, p_ref, d_ref, w_ref, o_ref):
        x = self_ref[...] + p_ref[0] + p_ref[1]
        deg = d_ref[0] + d_ref[1] + 1.0
        y = x / deg
        o_ref[...] = lax.dot_general(
            y, w_ref[...],
            dimension_numbers=(((1,), (1,)), ((), ())),
            preferred_element_type=jnp.float32)

    return pl.pallas_call(
        body,
        out_shape=jax.ShapeDtypeStruct((N, D), jnp.float32),
    )(self_feat, psum, pdeg, W)


def kernel(self_feat, nbr_feat, relation_src_indices, W):
    N, D = self_feat.shape
    E = nbr_feat.shape[0]
    NW = NC * NS
    e_per_tile = E // NW
    nblk_per_tile = e_per_tile // BLK
    assert e_per_tile * NW == E and nblk_per_tile * BLK == e_per_tile
    assert N % (NS * BLK) == 0
    deg_stripe = -(-(N // NS) // 128) * 128

    idx2d = relation_src_indices.astype(jnp.int32).reshape(NW * nblk_per_tile,
                                                           BLK)
    psum, pdeg = _sc_aggregate(nbr_feat, idx2d, N, nblk_per_tile, deg_stripe)
    pdeg3 = pdeg[:, :N].reshape(NC, N, 1)
    return _tc_finish(self_feat, psum, pdeg3, W)


# speedup vs baseline: 11.4395x; 1.0124x over previous
import functools

import jax
import jax.numpy as jnp
from jax import lax
from jax.experimental import pallas as pl
from jax.experimental.pallas import tpu as pltpu
from jax.experimental.pallas import tpu_sc as plsc

NC = 2
NS = 16
BLK = 100
NBUF = 3


def _sc_aggregate(nbr_feat, idx2d, zrows, consts, n_nodes, nblk_per_tile,
                  deg_stripe):
    E, D = nbr_feat.shape
    NW = NC * NS
    e_per_tile = E // NW
    rows_per_tile = n_nodes // NS
    deg_pad = NS * deg_stripe
    ngrp = nblk_per_tile // NBUF
    tail = nblk_per_tile - ngrp * NBUF

    mesh = plsc.VectorSubcoreMesh(core_axis_name="c", subcore_axis_name="s")

    @functools.partial(
        pl.kernel,
        mesh=mesh,
        compiler_params=pltpu.CompilerParams(use_tc_tiling_on_sc=False),
        out_type=[
            jax.ShapeDtypeStruct((NC, n_nodes, D), jnp.float32),
            jax.ShapeDtypeStruct((NC, deg_pad), jnp.float32),
        ],
        scratch_types=[
            pltpu.VMEM((nblk_per_tile, BLK), jnp.int32),
            pltpu.VMEM((NBUF, BLK, D), jnp.float32),
            pltpu.VMEM((128,), jnp.float32),
            pltpu.VMEM_SHARED((n_nodes, D), jnp.float32),
            pltpu.VMEM_SHARED((deg_pad,), jnp.float32),
            pltpu.SemaphoreType.DMA((NBUF,)),
            pltpu.SemaphoreType.DMA((NBUF,)),
            pltpu.SemaphoreType.DMA,
        ],
    )
    def k(nbr_hbm, idx_hbm, zrows_hbm, consts_hbm, out_sum, out_deg,
          idx_v, buf, ones_v, acc_sh, deg_sh, sem_g, sem_s, sem_d):
        c = lax.axis_index("c")
        s = lax.axis_index("s")
        wid = s * NC + c

        pltpu.sync_copy(idx_hbm.at[pl.ds(wid * nblk_per_tile, nblk_per_tile)],
                        idx_v)
        pltpu.sync_copy(consts_hbm.at[pl.ds(deg_stripe, 128)], ones_v)

        pltpu.sync_copy(zrows_hbm,
                        acc_sh.at[pl.ds(s * rows_per_tile, rows_per_tile)])
        pltpu.sync_copy(consts_hbm.at[pl.ds(0, deg_stripe)],
                        deg_sh.at[pl.ds(s * deg_stripe, deg_stripe)])

        plsc.subcore_barrier()

        ebase = wid * e_per_tile

        for b in range(NBUF):
            pltpu.async_copy(nbr_hbm.at[pl.ds(ebase + b * BLK, BLK)],
                             buf.at[b], sem_g.at[b])

        ones_blk = ones_v.at[pl.ds(0, BLK)]

        def grp_body(g, carry):
            base_blk = g * NBUF
            feat_descs = []
            deg_descs = []
            for b in range(NBUF):
                blk = base_blk + b
                pltpu.make_async_copy(nbr_hbm.at[pl.ds(ebase, BLK)],
                                      buf.at[b], sem_g.at[b]).wait()
                feat_descs.append(pltpu.async_copy(
                    buf.at[b], acc_sh.at[idx_v.at[blk]], sem_s.at[b],
                    add=True))
                deg_descs.append(pltpu.async_copy(
                    ones_blk, deg_sh.at[idx_v.at[blk]], sem_d, add=True))
            for b in range(NBUF):
                feat_descs[b].wait()
                deg_descs[b].wait()
                blk_next = base_blk + NBUF + b

                @pl.when(blk_next < nblk_per_tile)
                def _():
                    pltpu.async_copy(
                        nbr_hbm.at[pl.ds(ebase + blk_next * BLK, BLK)],
                        buf.at[b], sem_g.at[b])
            return carry

        lax.fori_loop(0, ngrp, grp_body, None)

        for t in range(tail):
            blk = ngrp * NBUF + t
            pltpu.make_async_copy(nbr_hbm.at[pl.ds(ebase, BLK)],
                                  buf.at[t], sem_g.at[t]).wait()
            pltpu.sync_copy(buf.at[t], acc_sh.at[idx_v.at[blk]], add=True)
            pltpu.sync_copy(ones_blk, deg_sh.at[idx_v.at[blk]], add=True)

        plsc.subcore_barrier()

        pltpu.sync_copy(acc_sh.at[pl.ds(s * rows_per_tile, rows_per_tile)],
                        out_sum.at[c, pl.ds(s * rows_per_tile, rows_per_tile)])
        pltpu.sync_copy(deg_sh.at[pl.ds(s * deg_stripe, deg_stripe)],
                        out_deg.at[c, pl.ds(s * deg_stripe, deg_stripe)])

    return k(nbr_feat, idx2d, zrows, consts)


def _tc_finish(self_feat, psum, pdeg, W):
    N, D = self_feat.shape

    def body(self_ref, p_ref, d_ref, w_ref, o_ref):
        x = self_ref[...] + p_ref[0] + p_ref[1]
        deg = d_ref[0] + d_ref[1] + 1.0
        y = x / deg
        o_ref[...] = lax.dot_general(
            y, w_ref[...],
            dimension_numbers=(((1,), (1,)), ((), ())),
            preferred_element_type=jnp.float32)

    return pl.pallas_call(
        body,
        out_shape=jax.ShapeDtypeStruct((N, D), jnp.float32),
    )(self_feat, psum, pdeg, W)


def kernel(self_feat, nbr_feat, relation_src_indices, W):
    N, D = self_feat.shape
    E = nbr_feat.shape[0]
    NW = NC * NS
    e_per_tile = E // NW
    nblk_per_tile = e_per_tile // BLK
    assert e_per_tile * NW == E and nblk_per_tile * BLK == e_per_tile
    assert N % NS == 0
    deg_stripe = -(-(N // NS) // 128) * 128

    idx2d = relation_src_indices.astype(jnp.int32).reshape(NW * nblk_per_tile,
                                                           BLK)
    zrows = jnp.zeros((N // NS, D), jnp.float32)
    consts = jnp.concatenate(
        [jnp.zeros((deg_stripe,), jnp.float32),
         jnp.ones((128,), jnp.float32)])
    psum, pdeg = _sc_aggregate(nbr_feat, idx2d, zrows, consts, N,
                               nblk_per_tile, deg_stripe)
    pdeg3 = pdeg[:, :N].reshape(NC, N, 1)
    return _tc_finish(self_feat, psum, pdeg3, W)


# speedup vs baseline: 11.5449x; 1.0092x over previous
import functools

import jax
import jax.numpy as jnp
from jax import lax
from jax.experimental import pallas as pl
from jax.experimental.pallas import tpu as pltpu
from jax.experimental.pallas import tpu_sc as plsc

NC = 2
NS = 16
BLK = 100
NBUF = 3


def _sc_aggregate(nbr_feat, idx2d, zrows, consts, n_nodes, nblk_per_tile,
                  deg_stripe):
    E, D = nbr_feat.shape
    NW = NC * NS
    e_per_tile = E // NW
    rows_per_tile = n_nodes // NS
    deg_pad = NS * deg_stripe
    ngrp = nblk_per_tile // NBUF
    tail = nblk_per_tile - ngrp * NBUF

    mesh = plsc.VectorSubcoreMesh(core_axis_name="c", subcore_axis_name="s")

    @functools.partial(
        pl.kernel,
        mesh=mesh,
        compiler_params=pltpu.CompilerParams(use_tc_tiling_on_sc=False),
        out_type=[
            jax.ShapeDtypeStruct((NC, n_nodes, D), jnp.float32),
            jax.ShapeDtypeStruct((NC, deg_pad), jnp.float32),
        ],
        scratch_types=[
            pltpu.VMEM((nblk_per_tile, BLK), jnp.int32),
            pltpu.VMEM((NBUF, BLK, D), jnp.float32),
            pltpu.VMEM((128,), jnp.float32),
            pltpu.VMEM_SHARED((n_nodes, D), jnp.float32),
            pltpu.VMEM_SHARED((deg_pad,), jnp.float32),
            pltpu.SemaphoreType.DMA((NBUF,)),
            pltpu.SemaphoreType.DMA((NBUF,)),
            pltpu.SemaphoreType.DMA,
        ],
    )
    def k(nbr_hbm, idx_hbm, zrows_hbm, consts_hbm, out_sum, out_deg,
          idx_v, buf, ones_v, acc_sh, deg_sh, sem_g, sem_s, sem_d):
        c = lax.axis_index("c")
        s = lax.axis_index("s")
        wid = s * NC + c

        ebase = wid * e_per_tile
        for b in range(NBUF):
            pltpu.async_copy(nbr_hbm.at[pl.ds(ebase + b * BLK, BLK)],
                             buf.at[b], sem_g.at[b])

        pltpu.sync_copy(idx_hbm.at[pl.ds(wid * nblk_per_tile, nblk_per_tile)],
                        idx_v)
        pltpu.sync_copy(consts_hbm.at[pl.ds(deg_stripe, 128)], ones_v)

        pltpu.sync_copy(zrows_hbm,
                        acc_sh.at[pl.ds(s * rows_per_tile, rows_per_tile)])
        pltpu.sync_copy(consts_hbm.at[pl.ds(0, deg_stripe)],
                        deg_sh.at[pl.ds(s * deg_stripe, deg_stripe)])

        plsc.subcore_barrier()

        ones_blk = ones_v.at[pl.ds(0, BLK)]

        def grp_body(g, carry):
            base_blk = g * NBUF
            feat_descs = []
            deg_descs = []
            for b in range(NBUF):
                blk = base_blk + b
                pltpu.make_async_copy(nbr_hbm.at[pl.ds(ebase, BLK)],
                                      buf.at[b], sem_g.at[b]).wait()
                feat_descs.append(pltpu.async_copy(
                    buf.at[b], acc_sh.at[idx_v.at[blk]], sem_s.at[b],
                    add=True))
                deg_descs.append(pltpu.async_copy(
                    ones_blk, deg_sh.at[idx_v.at[blk]], sem_d, add=True))
            for b in range(NBUF):
                feat_descs[b].wait()
                deg_descs[b].wait()
                blk_next = base_blk + NBUF + b

                @pl.when(blk_next < nblk_per_tile)
                def _():
                    pltpu.async_copy(
                        nbr_hbm.at[pl.ds(ebase + blk_next * BLK, BLK)],
                        buf.at[b], sem_g.at[b])
            return carry

        lax.fori_loop(0, ngrp, grp_body, None)

        for t in range(tail):
            blk = ngrp * NBUF + t
            pltpu.make_async_copy(nbr_hbm.at[pl.ds(ebase, BLK)],
                                  buf.at[t], sem_g.at[t]).wait()
            pltpu.sync_copy(buf.at[t], acc_sh.at[idx_v.at[blk]], add=True)
            pltpu.sync_copy(ones_blk, deg_sh.at[idx_v.at[blk]], add=True)

        plsc.subcore_barrier()

        pltpu.sync_copy(acc_sh.at[pl.ds(s * rows_per_tile, rows_per_tile)],
                        out_sum.at[c, pl.ds(s * rows_per_tile, rows_per_tile)])
        pltpu.sync_copy(deg_sh.at[pl.ds(s * deg_stripe, deg_stripe)],
                        out_deg.at[c, pl.ds(s * deg_stripe, deg_stripe)])

    return k(nbr_feat, idx2d, zrows, consts)


def _tc_finish(self_feat, psum, pdeg, W):
    N, D = self_feat.shape

    def body(self_ref, p_ref, d_ref, w_ref, o_ref):
        x = self_ref[...] + p_ref[0] + p_ref[1]
        deg = d_ref[0] + d_ref[1] + 1.0
        y = x / deg
        o_ref[...] = lax.dot_general(
            y, w_ref[...],
            dimension_numbers=(((1,), (1,)), ((), ())),
            preferred_element_type=jnp.float32)

    return pl.pallas_call(
        body,
        out_shape=jax.ShapeDtypeStruct((N, D), jnp.float32),
    )(self_feat, psum, pdeg, W)


def kernel(self_feat, nbr_feat, relation_src_indices, W):
    N, D = self_feat.shape
    E = nbr_feat.shape[0]
    NW = NC * NS
    e_per_tile = E // NW
    nblk_per_tile = e_per_tile // BLK
    assert e_per_tile * NW == E and nblk_per_tile * BLK == e_per_tile
    assert N % NS == 0
    deg_stripe = -(-(N // NS) // 128) * 128

    idx2d = relation_src_indices.astype(jnp.int32).reshape(NW * nblk_per_tile,
                                                           BLK)
    zrows = jnp.zeros((N // NS, D), jnp.float32)
    consts = jnp.concatenate(
        [jnp.zeros((deg_stripe,), jnp.float32),
         jnp.ones((128,), jnp.float32)])
    psum, pdeg = _sc_aggregate(nbr_feat, idx2d, zrows, consts, N,
                               nblk_per_tile, deg_stripe)
    pdeg3 = pdeg[:, :N].reshape(NC, N, 1)
    return _tc_finish(self_feat, psum, pdeg3, W)
